# Initial kernel scaffold; baseline (speedup 1.0000x reference)
import jax
import jax.numpy as jnp
from jax.experimental import pallas as pl


def kernel(boxes, scores, im_labels):
    raise NotImplementedError("write your pallas kernel here")


# speedup vs baseline: 7.9546x; 7.9546x over previous
import functools

import jax
import jax.numpy as jnp
from jax.experimental import pallas as pl

_N = 5000
_C = 20
_NP = 5120
_NEG = jnp.float32(-3.0e38)
_FAR = jnp.float32(1.0e8)


def _body(st_ref, bt_ref, il_ref, out_ref):
    st = st_ref[...]
    bt = bt_ref[...]

    m = jnp.max(st, axis=1, keepdims=True)
    col = jax.lax.broadcasted_iota(jnp.int32, st.shape, 1)
    idx = jnp.min(jnp.where(st == m, col, _NP), axis=1, keepdims=True)

    onehot = (col == idx).astype(jnp.float32)
    x1 = bt[0:1, :]
    y1 = bt[1:2, :]
    x2 = bt[2:3, :]
    y2 = bt[3:4, :]
    tx1 = jnp.sum(onehot * x1, axis=1, keepdims=True)
    ty1 = jnp.sum(onehot * y1, axis=1, keepdims=True)
    tx2 = jnp.sum(onehot * x2, axis=1, keepdims=True)
    ty2 = jnp.sum(onehot * y2, axis=1, keepdims=True)

    xx1 = jnp.maximum(x1, tx1)
    yy1 = jnp.maximum(y1, ty1)
    xx2 = jnp.minimum(x2, tx2)
    yy2 = jnp.minimum(y2, ty2)
    iw = xx2 - xx1 + 1.0
    ih = yy2 - yy1 + 1.0
    valid = ((iw > 0) & (ih > 0)).astype(jnp.float32)
    inter = iw * ih * valid
    area_n = (x2 - x1 + 1.0) * (y2 - y1 + 1.0)
    area_t = (tx2 - tx1 + 1.0) * (ty2 - ty1 + 1.0)
    iou = inter / (area_n + area_t - inter)

    cmask = (iou > 0.7).astype(jnp.float32)
    neglog = -jnp.log(jnp.clip(st, 1e-6, 1.0 - 1e-6))
    num = jnp.sum(neglog * cmask, axis=1, keepdims=True)
    den = jnp.maximum(jnp.sum(cmask, axis=1, keepdims=True), 1.0)
    per_class = num / den

    il = il_ref[...]
    loss = jnp.sum(per_class * il) / jnp.sum(il)
    out_ref[...] = jnp.broadcast_to(loss, (1, 1))


@jax.jit
def kernel(boxes, scores, im_labels):
    st = jnp.full((_C, _NP), _NEG, dtype=jnp.float32)
    st = st.at[:, :_N].set(scores.T)
    bt = jnp.full((4, _NP), _FAR, dtype=jnp.float32)
    bt = bt.at[:, :_N].set(boxes.T)
    ilt = im_labels.T.astype(jnp.float32)

    out = pl.pallas_call(
        _body,
        out_shape=jax.ShapeDtypeStruct((1, 1), jnp.float32),
    )(st, bt, ilt)
    return out[0, 0]


# speedup vs baseline: 16.2755x; 2.0461x over previous
import functools

import jax
import jax.numpy as jnp
from jax.experimental import pallas as pl

_N = 5000
_C = 20
_NP = 5120
_NEG = jnp.float32(-3.0e38)
_FAR = jnp.float32(1.0e8)


def _body(st_ref, bt_ref, il_ref, out_ref):
    st = st_ref[...]
    bt = bt_ref[...]

    m = jnp.max(st, axis=1, keepdims=True)
    col = jax.lax.broadcasted_iota(jnp.int32, st.shape, 1)
    idx = jnp.min(jnp.where(st == m, col, _N), axis=1, keepdims=True)

    onehot = (col == idx).astype(jnp.float32)
    x1 = bt[0:1, :]
    y1 = bt[1:2, :]
    x2 = bt[2:3, :]
    y2 = bt[3:4, :]
    tx1 = jnp.sum(onehot * x1, axis=1, keepdims=True)
    ty1 = jnp.sum(onehot * y1, axis=1, keepdims=True)
    tx2 = jnp.sum(onehot * x2, axis=1, keepdims=True)
    ty2 = jnp.sum(onehot * y2, axis=1, keepdims=True)

    xx1 = jnp.maximum(x1, tx1)
    yy1 = jnp.maximum(y1, ty1)
    xx2 = jnp.minimum(x2, tx2)
    yy2 = jnp.minimum(y2, ty2)
    iw = xx2 - xx1 + 1.0
    ih = yy2 - yy1 + 1.0
    valid = ((iw > 0) & (ih > 0)).astype(jnp.float32)
    inter = iw * ih * valid
    area_n = (x2 - x1 + 1.0) * (y2 - y1 + 1.0)
    area_t = (tx2 - tx1 + 1.0) * (ty2 - ty1 + 1.0)
    iou = inter / (area_n + area_t - inter)

    cmask = (iou > 0.7).astype(jnp.float32)
    neglog = -jnp.log(jnp.clip(st, 1e-6, 1.0 - 1e-6))
    num = jnp.sum(neglog * cmask, axis=1, keepdims=True)
    den = jnp.maximum(jnp.sum(cmask, axis=1, keepdims=True), 1.0)
    per_class = num / den

    il = il_ref[...]
    loss = jnp.sum(per_class * il) / jnp.sum(il)
    out_ref[...] = jnp.broadcast_to(loss, (1, 1))


@jax.jit
def kernel(boxes, scores, im_labels):
    st = scores.T
    bt = boxes.T
    ilt = im_labels.T.astype(jnp.float32)

    out = pl.pallas_call(
        _body,
        out_shape=jax.ShapeDtypeStruct((1, 1), jnp.float32),
    )(st, bt, ilt)
    return out[0, 0]
